# Initial kernel scaffold; baseline (speedup 1.0000x reference)
import jax
import jax.numpy as jnp
from jax.experimental import pallas as pl


def kernel(x, edge_index, weight, bias):
    raise NotImplementedError("write your pallas kernel here")


# speedup vs baseline: 10.9547x; 10.9547x over previous
import jax
import jax.numpy as jnp
from jax import lax
from jax.experimental import pallas as pl
from jax.experimental.pallas import tpu as pltpu
from jax.experimental.pallas import tpu_sc as plsc

N = 10000
E = 320000
D = 128
DH = D // 2

NC = 2
NS = 16
LANES = 16

CHUNK = 80

TAB_ROWS = 10240
TRASH = 10100

_mesh = plsc.VectorSubcoreMesh(
    core_axis_name="c", subcore_axis_name="s", num_cores=NC, num_subcores=NS)

_sc_params = pltpu.CompilerParams(use_tc_tiling_on_sc=False)

_f32 = jnp.float32


def _zero16():
    return jnp.broadcast_to(jnp.float32(0.0), (LANES,))


H_EPT = E // (NC * NS)
H_NCHUNKS = H_EPT // CHUNK


def _deg_body(rows_hbm, cols_hbm, out_hbm, table, ridx, cidx, ones_v, zbuf):
    cid = lax.axis_index("c")
    sid = lax.axis_index("s")

    lane = lax.iota(jnp.int32, LANES)
    one_hot = jnp.where(lane == 0, jnp.float32(1.0), jnp.float32(0.0))

    stripe = TAB_ROWS // NS

    def fill(i, _):
        zbuf[i, :] = _zero16()
        return 0
    lax.fori_loop(0, stripe, fill, 0)

    def ofill(i, _):
        ones_v[i, :] = one_hot
        return 0
    lax.fori_loop(0, CHUNK, ofill, 0)

    pltpu.sync_copy(zbuf, table.at[pl.ds(sid * stripe, stripe)])
    plsc.subcore_barrier()

    ebase = (cid * NS + sid) * H_EPT

    def chunk(g, _):
        off = pl.multiple_of(ebase + g * CHUNK, 8)
        pltpu.sync_copy(rows_hbm.at[pl.ds(off, CHUNK)], ridx)
        pltpu.sync_copy(cols_hbm.at[pl.ds(off, CHUNK)], cidx)
        for i in range(CHUNK // LANES):
            sl = pl.ds(i * LANES, LANES)
            r = ridx[sl]
            c = cidx[sl]
            ridx[sl] = jnp.where(r == c, jnp.int32(TRASH), r)
        pltpu.sync_copy(ones_v, table.at[ridx], add=True)
        return 0
    lax.fori_loop(0, H_NCHUNKS, chunk, 0)

    plsc.subcore_barrier()
    pltpu.sync_copy(table.at[pl.ds(sid * stripe, stripe)],
                    out_hbm.at[cid, pl.ds(sid * stripe, stripe)])


_deg_call = pl.kernel(
    _deg_body,
    out_type=jax.ShapeDtypeStruct((NC, TAB_ROWS, LANES), _f32),
    mesh=_mesh,
    scratch_types=[
        pltpu.VMEM_SHARED((TAB_ROWS, LANES), _f32),
        pltpu.VMEM((CHUNK,), jnp.int32),
        pltpu.VMEM((CHUNK,), jnp.int32),
        pltpu.VMEM((CHUNK, LANES), _f32),
        pltpu.VMEM((TAB_ROWS // NS, LANES), _f32),
    ],
    compiler_params=_sc_params,
)


A_EPT = E // NS
A_NCHUNKS = A_EPT // CHUNK


def _agg_body(rows_hbm, cols_hbm, lo_hbm, hi_hbm, out_hbm,
              table, ridx, cidx, rowbuf, zbuf, sem):
    cid = lax.axis_index("c")
    sid = lax.axis_index("s")

    stripe = TAB_ROWS // NS

    def fill(i, _):
        for j in range(DH // LANES):
            zbuf[i, pl.ds(j * LANES, LANES)] = _zero16()
        return 0
    lax.fori_loop(0, stripe, fill, 0)

    pltpu.sync_copy(zbuf, table.at[pl.ds(sid * stripe, stripe)])
    plsc.subcore_barrier()

    ebase = sid * A_EPT

    def chunk(g, _):
        off = pl.multiple_of(ebase + g * CHUNK, 8)
        pltpu.sync_copy(rows_hbm.at[pl.ds(off, CHUNK)], ridx)
        pltpu.sync_copy(cols_hbm.at[pl.ds(off, CHUNK)], cidx)
        for i in range(CHUNK // LANES):
            sl = pl.ds(i * LANES, LANES)
            r = ridx[sl]
            c = cidx[sl]
            cidx[sl] = jnp.where(r == c, jnp.int32(TRASH), c)

        @pl.when(cid == 0)
        def _():
            pltpu.async_copy(lo_hbm.at[ridx], rowbuf, sem).wait()

        @pl.when(cid == 1)
        def _():
            pltpu.async_copy(hi_hbm.at[ridx], rowbuf, sem).wait()

        pltpu.sync_copy(rowbuf, table.at[cidx], add=True)
        return 0
    lax.fori_loop(0, A_NCHUNKS, chunk, 0)

    plsc.subcore_barrier()
    pltpu.sync_copy(table.at[pl.ds(sid * stripe, stripe)],
                    out_hbm.at[cid, pl.ds(sid * stripe, stripe)])


_agg_call = pl.kernel(
    _agg_body,
    out_type=jax.ShapeDtypeStruct((NC, TAB_ROWS, DH), _f32),
    mesh=_mesh,
    scratch_types=[
        pltpu.VMEM_SHARED((TAB_ROWS, DH), _f32),
        pltpu.VMEM((CHUNK,), jnp.int32),
        pltpu.VMEM((CHUNK,), jnp.int32),
        pltpu.VMEM((CHUNK, DH), _f32),
        pltpu.VMEM((TAB_ROWS // NS, DH), _f32),
        pltpu.SemaphoreType.DMA,
    ],
    compiler_params=_sc_params,
)


_BLK = 1000


def _deg_from_hist(h):
    return 1.0 + jnp.sum(h[0], axis=-1) + jnp.sum(h[1], axis=-1)


def _mm_body(x_ref, w_ref, h_ref, xw_ref, xws_ref):
    xw = jnp.dot(x_ref[...], w_ref[...], preferred_element_type=_f32)
    dis = lax.rsqrt(_deg_from_hist(h_ref[...]))
    xw_ref[...] = xw
    xws_ref[...] = xw * dis[:, None]


def _mm_call(x, weight, hist):
    return pl.pallas_call(
        _mm_body,
        grid=(N // _BLK,),
        in_specs=[
            pl.BlockSpec((_BLK, D), lambda i: (i, 0)),
            pl.BlockSpec((D, D), lambda i: (0, 0)),
            pl.BlockSpec((NC, _BLK, LANES), lambda i: (0, i, 0)),
        ],
        out_specs=[
            pl.BlockSpec((_BLK, D), lambda i: (i, 0)),
            pl.BlockSpec((_BLK, D), lambda i: (i, 0)),
        ],
        out_shape=[
            jax.ShapeDtypeStruct((N, D), _f32),
            jax.ShapeDtypeStruct((N, D), _f32),
        ],
    )(x, weight, hist)


def _fin_body(a_ref, xw_ref, h_ref, b_ref, o_ref):
    deg = _deg_from_hist(h_ref[...])
    dis = lax.rsqrt(deg)
    s2 = 1.0 + 1.0 / deg
    aggr = jnp.concatenate([a_ref[0], a_ref[1]], axis=-1)
    o_ref[...] = aggr * dis[:, None] + xw_ref[...] * s2[:, None] + b_ref[...]


def _fin_call(aggr, xw, hist, bias2d):
    return pl.pallas_call(
        _fin_body,
        grid=(N // _BLK,),
        in_specs=[
            pl.BlockSpec((NC, _BLK, DH), lambda i: (0, i, 0)),
            pl.BlockSpec((_BLK, D), lambda i: (i, 0)),
            pl.BlockSpec((NC, _BLK, LANES), lambda i: (0, i, 0)),
            pl.BlockSpec((1, D), lambda i: (0, 0)),
        ],
        out_specs=pl.BlockSpec((_BLK, D), lambda i: (i, 0)),
        out_shape=jax.ShapeDtypeStruct((N, D), _f32),
    )(aggr, xw, hist, bias2d)


def kernel(x, edge_index, weight, bias):
    rows = edge_index[0]
    cols = edge_index[1]
    hist = _deg_call(rows, cols)[:, :N, :]
    xw, xws = _mm_call(x, weight, hist)
    xws_lo = lax.slice(xws, (0, 0), (N, DH))
    xws_hi = lax.slice(xws, (0, DH), (N, D))
    aggr = _agg_call(rows, cols, xws_lo, xws_hi)[:, :N, :]
    return _fin_call(aggr, xw, hist, bias[None, :])


# speedup vs baseline: 34.5526x; 3.1541x over previous
import jax
import jax.numpy as jnp
from jax import lax
from jax.experimental import pallas as pl
from jax.experimental.pallas import tpu as pltpu
from jax.experimental.pallas import tpu_sc as plsc

N = 10000
E = 320000
D = 128
DH = D // 2

NC = 2
NS = 16
LANES = 16

CHUNK = 80
EROWS = E // CHUNK

TAB_ROWS = 10240
STRIPE = TAB_ROWS // NS
TRASH = 10100

_mesh = plsc.VectorSubcoreMesh(
    core_axis_name="c", subcore_axis_name="s", num_cores=NC, num_subcores=NS)

_sc_params = pltpu.CompilerParams(use_tc_tiling_on_sc=False)

_f32 = jnp.float32


def _zero16():
    return jnp.broadcast_to(jnp.float32(0.0), (LANES,))


H_ROWS = EROWS // (NC * NS)
H_SUP = 25
H_NSUP = H_ROWS // H_SUP


def _deg_body(rows_hbm, cols_hbm, out_hbm, table, ridx, cidx, ones_v, zbuf,
              ssem):
    cid = lax.axis_index("c")
    sid = lax.axis_index("s")

    lane = lax.iota(jnp.int32, LANES)
    one_hot = jnp.where(lane == 0, jnp.float32(1.0), jnp.float32(0.0))

    def zfill(i, _):
        zbuf[i, :] = _zero16()
        return 0
    lax.fori_loop(0, STRIPE, zfill, 0)

    def ofill(i, _):
        ones_v[i, :] = one_hot
        return 0
    lax.fori_loop(0, CHUNK, ofill, 0)

    pltpu.sync_copy(zbuf, table.at[pl.ds(sid * STRIPE, STRIPE)])
    plsc.subcore_barrier()

    rbase = (cid * NS + sid) * H_ROWS
    pltpu.sync_copy(rows_hbm.at[pl.ds(rbase, H_ROWS)], ridx)
    pltpu.sync_copy(cols_hbm.at[pl.ds(rbase, H_ROWS)], cidx)

    def fix(row, _):
        for i in range(CHUNK // LANES):
            sl = pl.ds(i * LANES, LANES)
            r = ridx[row, sl]
            c = cidx[row, sl]
            ridx[row, sl] = jnp.where(r == c, jnp.int32(TRASH), r)
        return 0
    lax.fori_loop(0, H_ROWS, fix, 0)

    def drain(s):
        for j in range(H_SUP):
            pltpu.make_async_copy(
                ones_v, table.at[ridx.at[s * H_SUP + j]], ssem).wait()

    def loop(s, _):
        for j in range(H_SUP):
            pltpu.async_copy(
                ones_v, table.at[ridx.at[s * H_SUP + j]], ssem, add=True)

        @pl.when(s > 0)
        def _():
            drain(s - 1)
        return 0
    lax.fori_loop(0, H_NSUP, loop, 0)
    drain(H_NSUP - 1)

    plsc.subcore_barrier()
    pltpu.sync_copy(table.at[pl.ds(sid * STRIPE, STRIPE)],
                    out_hbm.at[cid, pl.ds(sid * STRIPE, STRIPE)])


_deg_call = pl.kernel(
    _deg_body,
    out_type=jax.ShapeDtypeStruct((NC, TAB_ROWS, LANES), _f32),
    mesh=_mesh,
    scratch_types=[
        pltpu.VMEM_SHARED((TAB_ROWS, LANES), _f32),
        pltpu.VMEM((H_ROWS, CHUNK), jnp.int32),
        pltpu.VMEM((H_ROWS, CHUNK), jnp.int32),
        pltpu.VMEM((CHUNK, LANES), _f32),
        pltpu.VMEM((STRIPE, LANES), _f32),
        pltpu.SemaphoreType.DMA,
    ],
    compiler_params=_sc_params,
)


A_ROWS = EROWS // NS
A_PH = 2
A_PROWS = A_ROWS // A_PH
A_SUP = 5
A_NSUP = A_PROWS // A_SUP


def _agg_body(rows_hbm, cols_hbm, xcat_hbm, out_hbm,
              table, ridx, cidx, rbufA, rbufB, gsem, ssem):
    cid = lax.axis_index("c")
    sid = lax.axis_index("s")

    def zfill(i, _):
        for j in range(DH // LANES):
            rbufA[0, i, pl.ds(j * LANES, LANES)] = _zero16()
        return 0
    lax.fori_loop(0, CHUNK, zfill, 0)

    for k in range(STRIPE // CHUNK):
        pltpu.sync_copy(rbufA.at[0],
                        table.at[pl.ds(sid * STRIPE + k * CHUNK, CHUNK)])
    plsc.subcore_barrier()

    roff = cid * N

    def fix(row, _):
        for i in range(CHUNK // LANES):
            sl = pl.ds(i * LANES, LANES)
            r = ridx[row, sl]
            c = cidx[row, sl]
            cidx[row, sl] = jnp.where(r == c, jnp.int32(TRASH), c)
            ridx[row, sl] = r + roff
        return 0

    def drain_scatter(s, rbuf):
        for j in range(A_SUP):
            pltpu.make_async_copy(
                rbuf.at[j], table.at[cidx.at[s * A_SUP + j]], ssem).wait()

    def do_super(s, rbuf, rbuf_prev):
        gds = [
            pltpu.async_copy(
                xcat_hbm.at[ridx.at[s * A_SUP + j]], rbuf.at[j], gsem)
            for j in range(A_SUP)
        ]

        @pl.when(s > 0)
        def _():
            drain_scatter(s - 1, rbuf_prev)

        for d in gds:
            d.wait()
        for j in range(A_SUP):
            pltpu.async_copy(
                rbuf.at[j], table.at[cidx.at[s * A_SUP + j]], ssem, add=True)

    for ph in range(A_PH):
        rbase = sid * A_ROWS + ph * A_PROWS
        pltpu.sync_copy(rows_hbm.at[pl.ds(rbase, A_PROWS)], ridx)
        pltpu.sync_copy(cols_hbm.at[pl.ds(rbase, A_PROWS)], cidx)
        lax.fori_loop(0, A_PROWS, fix, 0)

        def pair(h, _):
            do_super(2 * h, rbufA, rbufB)
            do_super(2 * h + 1, rbufB, rbufA)
            return 0
        lax.fori_loop(0, (A_NSUP - 1) // 2, pair, 0)
        last = A_NSUP - 1
        gds = [
            pltpu.async_copy(
                xcat_hbm.at[ridx.at[last * A_SUP + j]], rbufA.at[j], gsem)
            for j in range(A_SUP)
        ]
        drain_scatter(last - 1, rbufB)
        for d in gds:
            d.wait()
        for j in range(A_SUP):
            pltpu.async_copy(
                rbufA.at[j], table.at[cidx.at[last * A_SUP + j]], ssem,
                add=True)
        drain_scatter(last, rbufA)

    plsc.subcore_barrier()
    pltpu.sync_copy(table.at[pl.ds(sid * STRIPE, STRIPE)],
                    out_hbm.at[cid, pl.ds(sid * STRIPE, STRIPE)])


_agg_call = pl.kernel(
    _agg_body,
    out_type=jax.ShapeDtypeStruct((NC, TAB_ROWS, DH), _f32),
    mesh=_mesh,
    scratch_types=[
        pltpu.VMEM_SHARED((TAB_ROWS, DH), _f32),
        pltpu.VMEM((A_PROWS, CHUNK), jnp.int32),
        pltpu.VMEM((A_PROWS, CHUNK), jnp.int32),
        pltpu.VMEM((A_SUP, CHUNK, DH), _f32),
        pltpu.VMEM((A_SUP, CHUNK, DH), _f32),
        pltpu.SemaphoreType.DMA,
        pltpu.SemaphoreType.DMA,
    ],
    compiler_params=_sc_params,
)


_BLK = 1000


def _deg_from_hist(h):
    return 1.0 + jnp.sum(h[0], axis=-1) + jnp.sum(h[1], axis=-1)


def _mm_body(x_ref, w_ref, h_ref, xw_ref, xws_ref):
    xw = jnp.dot(x_ref[...], w_ref[...], preferred_element_type=_f32)
    dis = lax.rsqrt(_deg_from_hist(h_ref[...]))
    xws = xw * dis[:, None]
    xw_ref[...] = xw
    xws_ref[0] = xws[:, :DH]
    xws_ref[1] = xws[:, DH:]


def _mm_call(x, weight, hist):
    return pl.pallas_call(
        _mm_body,
        grid=(N // _BLK,),
        in_specs=[
            pl.BlockSpec((_BLK, D), lambda i: (i, 0)),
            pl.BlockSpec((D, D), lambda i: (0, 0)),
            pl.BlockSpec((NC, _BLK, LANES), lambda i: (0, i, 0)),
        ],
        out_specs=[
            pl.BlockSpec((_BLK, D), lambda i: (i, 0)),
            pl.BlockSpec((2, _BLK, DH), lambda i: (0, i, 0)),
        ],
        out_shape=[
            jax.ShapeDtypeStruct((N, D), _f32),
            jax.ShapeDtypeStruct((2, N, DH), _f32),
        ],
    )(x, weight, hist)


def _fin_body(a_ref, xw_ref, h_ref, b_ref, o_ref):
    deg = _deg_from_hist(h_ref[...])
    dis = lax.rsqrt(deg)
    s2 = 1.0 + 1.0 / deg
    aggr = jnp.concatenate([a_ref[0], a_ref[1]], axis=-1)
    o_ref[...] = aggr * dis[:, None] + xw_ref[...] * s2[:, None] + b_ref[...]


def _fin_call(aggr, xw, hist, bias2d):
    return pl.pallas_call(
        _fin_body,
        grid=(N // _BLK,),
        in_specs=[
            pl.BlockSpec((NC, _BLK, DH), lambda i: (0, i, 0)),
            pl.BlockSpec((_BLK, D), lambda i: (i, 0)),
            pl.BlockSpec((NC, _BLK, LANES), lambda i: (0, i, 0)),
            pl.BlockSpec((1, D), lambda i: (0, 0)),
        ],
        out_specs=pl.BlockSpec((_BLK, D), lambda i: (i, 0)),
        out_shape=jax.ShapeDtypeStruct((N, D), _f32),
    )(aggr, xw, hist, bias2d)


def kernel(x, edge_index, weight, bias):
    rows2 = edge_index[0].reshape(EROWS, CHUNK)
    cols2 = edge_index[1].reshape(EROWS, CHUNK)
    hist = _deg_call(rows2, cols2)[:, :N, :]
    xw, xws2 = _mm_call(x, weight, hist)
    xcat = xws2.reshape(2 * N, DH)
    aggr = _agg_call(rows2, cols2, xcat)[:, :N, :]
    return _fin_call(aggr, xw, hist, bias[None, :])


# speedup vs baseline: 35.7274x; 1.0340x over previous
import jax
import jax.numpy as jnp
from jax import lax
from jax.experimental import pallas as pl
from jax.experimental.pallas import tpu as pltpu
from jax.experimental.pallas import tpu_sc as plsc

N = 10000
E = 320000
D = 128
DH = D // 2

NC = 2
NS = 16
LANES = 16

CHUNK = 80
EROWS = E // CHUNK

TAB_ROWS = 10240
STRIPE = TAB_ROWS // NS
TRASH = 10100

_mesh = plsc.VectorSubcoreMesh(
    core_axis_name="c", subcore_axis_name="s", num_cores=NC, num_subcores=NS)

_sc_params = pltpu.CompilerParams(use_tc_tiling_on_sc=False)

_f32 = jnp.float32


def _zero16():
    return jnp.broadcast_to(jnp.float32(0.0), (LANES,))


H_ROWS = EROWS // (NC * NS)
H_SUP = 25
H_NSUP = H_ROWS // H_SUP


def _deg_body(ei_hbm, out_hbm, table, ridx, cidx, ones_v, zbuf, ssem):
    cid = lax.axis_index("c")
    sid = lax.axis_index("s")

    lane = lax.iota(jnp.int32, LANES)
    one_hot = jnp.where(lane == 0, jnp.float32(1.0), jnp.float32(0.0))

    def zfill(i, _):
        zbuf[i, :] = _zero16()
        return 0
    lax.fori_loop(0, STRIPE, zfill, 0)

    def ofill(i, _):
        ones_v[i, :] = one_hot
        return 0
    lax.fori_loop(0, CHUNK, ofill, 0)

    pltpu.sync_copy(zbuf, table.at[pl.ds(sid * STRIPE, STRIPE)])
    plsc.subcore_barrier()

    rbase = (cid * NS + sid) * H_ROWS
    pltpu.sync_copy(ei_hbm.at[0, pl.ds(rbase, H_ROWS)], ridx)
    pltpu.sync_copy(ei_hbm.at[1, pl.ds(rbase, H_ROWS)], cidx)

    def fix(row, _):
        for i in range(CHUNK // LANES):
            sl = pl.ds(i * LANES, LANES)
            r = ridx[row, sl]
            c = cidx[row, sl]
            ridx[row, sl] = jnp.where(r == c, jnp.int32(TRASH), r)
        return 0
    lax.fori_loop(0, H_ROWS, fix, 0)

    def drain(s):
        for j in range(H_SUP):
            pltpu.make_async_copy(
                ones_v, table.at[ridx.at[s * H_SUP + j]], ssem).wait()

    def loop(s, _):
        for j in range(H_SUP):
            pltpu.async_copy(
                ones_v, table.at[ridx.at[s * H_SUP + j]], ssem, add=True)

        @pl.when(s > 0)
        def _():
            drain(s - 1)
        return 0
    lax.fori_loop(0, H_NSUP, loop, 0)
    drain(H_NSUP - 1)

    plsc.subcore_barrier()
    pltpu.sync_copy(table.at[pl.ds(sid * STRIPE, STRIPE)],
                    out_hbm.at[cid, pl.ds(sid * STRIPE, STRIPE)])


_deg_call = pl.kernel(
    _deg_body,
    out_type=jax.ShapeDtypeStruct((NC, TAB_ROWS, LANES), _f32),
    mesh=_mesh,
    scratch_types=[
        pltpu.VMEM_SHARED((TAB_ROWS, LANES), _f32),
        pltpu.VMEM((H_ROWS, CHUNK), jnp.int32),
        pltpu.VMEM((H_ROWS, CHUNK), jnp.int32),
        pltpu.VMEM((CHUNK, LANES), _f32),
        pltpu.VMEM((STRIPE, LANES), _f32),
        pltpu.SemaphoreType.DMA,
    ],
    compiler_params=_sc_params,
)


A_ROWS = EROWS // NS
A_PH = 2
A_PROWS = A_ROWS // A_PH
A_SUP = 5
A_NSUP = A_PROWS // A_SUP


def _agg_body(ei_hbm, xcat_hbm, xw2_hbm, dis_hbm, s2_hbm, bias2_hbm, out_hbm,
              table, ridx, cidx, rbufA, rbufB,
              ebuf, xbuf, dbuf, sbuf, bias_v, gsem, ssem):
    cid = lax.axis_index("c")
    sid = lax.axis_index("s")

    def zfill(i, _):
        for j in range(DH // LANES):
            rbufA[0, i, pl.ds(j * LANES, LANES)] = _zero16()
        return 0
    lax.fori_loop(0, CHUNK, zfill, 0)

    for k in range(STRIPE // CHUNK):
        pltpu.sync_copy(rbufA.at[0],
                        table.at[pl.ds(sid * STRIPE + k * CHUNK, CHUNK)])
    plsc.subcore_barrier()

    roff = cid * TAB_ROWS

    def fix(row, _):
        for i in range(CHUNK // LANES):
            sl = pl.ds(i * LANES, LANES)
            r = ridx[row, sl]
            c = cidx[row, sl]
            cidx[row, sl] = jnp.where(r == c, jnp.int32(TRASH), c)
            ridx[row, sl] = r + roff
        return 0

    def drain_scatter(s, rbuf):
        for j in range(A_SUP):
            pltpu.make_async_copy(
                rbuf.at[j], table.at[cidx.at[s * A_SUP + j]], ssem).wait()

    def do_super(s, rbuf, rbuf_prev):
        gds = [
            pltpu.async_copy(
                xcat_hbm.at[ridx.at[s * A_SUP + j]], rbuf.at[j], gsem)
            for j in range(A_SUP)
        ]

        @pl.when(s > 0)
        def _():
            drain_scatter(s - 1, rbuf_prev)

        for d in gds:
            d.wait()
        for j in range(A_SUP):
            pltpu.async_copy(
                rbuf.at[j], table.at[cidx.at[s * A_SUP + j]], ssem, add=True)

    for ph in range(A_PH):
        rbase = sid * A_ROWS + ph * A_PROWS
        pltpu.sync_copy(ei_hbm.at[0, pl.ds(rbase, A_PROWS)], ridx)
        pltpu.sync_copy(ei_hbm.at[1, pl.ds(rbase, A_PROWS)], cidx)
        lax.fori_loop(0, A_PROWS, fix, 0)

        def pair(h, _):
            do_super(2 * h, rbufA, rbufB)
            do_super(2 * h + 1, rbufB, rbufA)
            return 0
        lax.fori_loop(0, (A_NSUP - 1) // 2, pair, 0)
        last = A_NSUP - 1
        gds = [
            pltpu.async_copy(
                xcat_hbm.at[ridx.at[last * A_SUP + j]], rbufA.at[j], gsem)
            for j in range(A_SUP)
        ]
        drain_scatter(last - 1, rbufB)
        for d in gds:
            d.wait()
        for j in range(A_SUP):
            pltpu.async_copy(
                rbufA.at[j], table.at[cidx.at[last * A_SUP + j]], ssem,
                add=True)
        drain_scatter(last, rbufA)

    plsc.subcore_barrier()

    pltpu.sync_copy(bias2_hbm.at[cid], bias_v)

    def piece(p, _):
        start = sid * STRIPE + p * CHUNK

        @pl.when(start < N)
        def _():
            pltpu.sync_copy(table.at[pl.ds(start, CHUNK)], ebuf)
            pltpu.sync_copy(xw2_hbm.at[cid, pl.ds(start, CHUNK)], xbuf)
            pltpu.sync_copy(dis_hbm.at[pl.ds(start, CHUNK)], dbuf)
            pltpu.sync_copy(s2_hbm.at[pl.ds(start, CHUNK)], sbuf)
            bvs = [bias_v[pl.ds(q * LANES, LANES)] for q in range(DH // LANES)]
            for g in range(CHUNK // LANES):
                dv = dbuf[pl.ds(g * LANES, LANES)]
                sv = sbuf[pl.ds(g * LANES, LANES)]
                for j in range(LANES):
                    r = g * LANES + j
                    d = jnp.broadcast_to(dv[j], (LANES,))
                    s = jnp.broadcast_to(sv[j], (LANES,))
                    for q in range(DH // LANES):
                        sl = pl.ds(q * LANES, LANES)
                        ebuf[r, sl] = (d * ebuf[r, sl] + s * xbuf[r, sl]
                                       + bvs[q])
            pltpu.sync_copy(
                ebuf, out_hbm.at[pl.ds(start, CHUNK), pl.ds(cid * DH, DH)])
        return 0
    lax.fori_loop(0, STRIPE // CHUNK, piece, 0)


_agg_call = pl.kernel(
    _agg_body,
    out_type=jax.ShapeDtypeStruct((N, D), _f32),
    mesh=_mesh,
    scratch_types=[
        pltpu.VMEM_SHARED((TAB_ROWS, DH), _f32),
        pltpu.VMEM((A_PROWS, CHUNK), jnp.int32),
        pltpu.VMEM((A_PROWS, CHUNK), jnp.int32),
        pltpu.VMEM((A_SUP, CHUNK, DH), _f32),
        pltpu.VMEM((A_SUP, CHUNK, DH), _f32),
        pltpu.VMEM((CHUNK, DH), _f32),
        pltpu.VMEM((CHUNK, DH), _f32),
        pltpu.VMEM((CHUNK,), _f32),
        pltpu.VMEM((CHUNK,), _f32),
        pltpu.VMEM((DH,), _f32),
        pltpu.SemaphoreType.DMA,
        pltpu.SemaphoreType.DMA,
    ],
    compiler_params=_sc_params,
)


_BLK = 1000


def _mm_body(x_ref, w_ref, h_ref, xws_ref, xw_ref, dis_ref, s2_ref):
    xw = jnp.dot(x_ref[...], w_ref[...], preferred_element_type=_f32)
    deg = 1.0 + jnp.sum(h_ref[0], axis=-1) + jnp.sum(h_ref[1], axis=-1)
    dis = lax.rsqrt(deg)
    xws = xw * dis[:, None]
    xws_ref[0] = xws[:, :DH]
    xws_ref[1] = xws[:, DH:]
    xw_ref[0] = xw[:, :DH]
    xw_ref[1] = xw[:, DH:]
    dis_ref[...] = dis[None, None, :]
    s2_ref[...] = (1.0 + 1.0 / deg)[None, None, :]


def _mm_call(x, weight, hist):
    return pl.pallas_call(
        _mm_body,
        grid=(N // _BLK,),
        in_specs=[
            pl.BlockSpec((_BLK, D), lambda i: (i, 0)),
            pl.BlockSpec((D, D), lambda i: (0, 0)),
            pl.BlockSpec((NC, _BLK, LANES), lambda i: (0, i, 0)),
        ],
        out_specs=[
            pl.BlockSpec((2, _BLK, DH), lambda i: (0, i, 0)),
            pl.BlockSpec((2, _BLK, DH), lambda i: (0, i, 0)),
            pl.BlockSpec((1, 1, _BLK), lambda i: (i, 0, 0)),
            pl.BlockSpec((1, 1, _BLK), lambda i: (i, 0, 0)),
        ],
        out_shape=[
            jax.ShapeDtypeStruct((2, TAB_ROWS, DH), _f32),
            jax.ShapeDtypeStruct((2, TAB_ROWS, DH), _f32),
            jax.ShapeDtypeStruct((N // _BLK, 1, _BLK), _f32),
            jax.ShapeDtypeStruct((N // _BLK, 1, _BLK), _f32),
        ],
    )(x, weight, hist)


def kernel(x, edge_index, weight, bias):
    ei3 = edge_index.reshape(2, EROWS, CHUNK)
    hist = _deg_call(ei3)
    xws2, xw2, dis_p, s2_p = _mm_call(x, weight, hist)
    xcat = xws2.reshape(2 * TAB_ROWS, DH)
    bias2 = bias.reshape(2, DH)
    return _agg_call(ei3, xcat, xw2, dis_p.reshape(N), s2_p.reshape(N), bias2)


# speedup vs baseline: 36.5387x; 1.0227x over previous
import jax
import jax.numpy as jnp
from jax import lax
from jax.experimental import pallas as pl
from jax.experimental.pallas import tpu as pltpu
from jax.experimental.pallas import tpu_sc as plsc

N = 10000
E = 320000
D = 128
DH = D // 2

NC = 2
NS = 16
LANES = 16

CHUNK = 80
EROWS = E // CHUNK

TAB_ROWS = 10240
STRIPE = TAB_ROWS // NS
TRASH = 10100

_mesh = plsc.VectorSubcoreMesh(
    core_axis_name="c", subcore_axis_name="s", num_cores=NC, num_subcores=NS)

_sc_params = pltpu.CompilerParams(use_tc_tiling_on_sc=False)

_f32 = jnp.float32


def _zero16():
    return jnp.broadcast_to(jnp.float32(0.0), (LANES,))


H_ROWS = EROWS // (NC * NS)
H_SUP = 25
H_NSUP = H_ROWS // H_SUP


def _deg_body(ei_hbm, out_hbm, table, ridx, cidx, ones_v, zbuf, ssem):
    cid = lax.axis_index("c")
    sid = lax.axis_index("s")

    lane = lax.iota(jnp.int32, LANES)
    one_hot = jnp.where(lane == 0, jnp.float32(1.0), jnp.float32(0.0))

    def zfill(i, _):
        zbuf[i, :] = _zero16()
        return 0
    lax.fori_loop(0, STRIPE, zfill, 0)

    def ofill(i, _):
        ones_v[i, :] = one_hot
        return 0
    lax.fori_loop(0, CHUNK, ofill, 0)

    pltpu.sync_copy(zbuf, table.at[pl.ds(sid * STRIPE, STRIPE)])
    plsc.subcore_barrier()

    rbase = (cid * NS + sid) * H_ROWS
    pltpu.sync_copy(ei_hbm.at[0, pl.ds(rbase, H_ROWS)], ridx)
    pltpu.sync_copy(ei_hbm.at[1, pl.ds(rbase, H_ROWS)], cidx)

    def fix_sup(s):
        def fix(row, _):
            for i in range(CHUNK // LANES):
                sl = pl.ds(i * LANES, LANES)
                r = ridx[row, sl]
                c = cidx[row, sl]
                ridx[row, sl] = jnp.where(r == c, jnp.int32(TRASH), r)
            return 0
        lax.fori_loop(s * H_SUP, (s + 1) * H_SUP, fix, 0)

    def drain(s):
        for j in range(H_SUP):
            pltpu.make_async_copy(
                ones_v, table.at[ridx.at[s * H_SUP + j]], ssem).wait()

    fix_sup(0)

    def loop(s, _):
        for j in range(H_SUP):
            pltpu.async_copy(
                ones_v, table.at[ridx.at[s * H_SUP + j]], ssem, add=True)

        @pl.when(s < H_NSUP - 1)
        def _():
            fix_sup(s + 1)

        @pl.when(s > 0)
        def _():
            drain(s - 1)
        return 0
    lax.fori_loop(0, H_NSUP, loop, 0)
    drain(H_NSUP - 1)

    plsc.subcore_barrier()
    pltpu.sync_copy(table.at[pl.ds(sid * STRIPE, STRIPE)],
                    out_hbm.at[cid, pl.ds(sid * STRIPE, STRIPE)])


_deg_call = pl.kernel(
    _deg_body,
    out_type=jax.ShapeDtypeStruct((NC, TAB_ROWS, LANES), _f32),
    mesh=_mesh,
    scratch_types=[
        pltpu.VMEM_SHARED((TAB_ROWS, LANES), _f32),
        pltpu.VMEM((H_ROWS, CHUNK), jnp.int32),
        pltpu.VMEM((H_ROWS, CHUNK), jnp.int32),
        pltpu.VMEM((CHUNK, LANES), _f32),
        pltpu.VMEM((STRIPE, LANES), _f32),
        pltpu.SemaphoreType.DMA,
    ],
    compiler_params=_sc_params,
)


A_ROWS = EROWS // NS
A_PH = 2
A_PROWS = A_ROWS // A_PH
A_SUP = 5
A_NSUP = A_PROWS // A_SUP


def _agg_body(ei_hbm, xcat_hbm, y2_hbm, dis_hbm, out_hbm,
              table, ridx, cidx, rbufA, rbufB,
              ebuf, xbuf, dbuf, gsem, ssem):
    cid = lax.axis_index("c")
    sid = lax.axis_index("s")

    def zfill(i, _):
        for j in range(DH // LANES):
            rbufA[0, i, pl.ds(j * LANES, LANES)] = _zero16()
        return 0
    lax.fori_loop(0, CHUNK, zfill, 0)

    for k in range(STRIPE // CHUNK):
        pltpu.sync_copy(rbufA.at[0],
                        table.at[pl.ds(sid * STRIPE + k * CHUNK, CHUNK)])
    plsc.subcore_barrier()

    roff = cid * TAB_ROWS

    def fix_sup(s):
        def fix(row, _):
            for i in range(CHUNK // LANES):
                sl = pl.ds(i * LANES, LANES)
                r = ridx[row, sl]
                c = cidx[row, sl]
                cidx[row, sl] = jnp.where(r == c, jnp.int32(TRASH), c)
                ridx[row, sl] = r + roff
            return 0
        lax.fori_loop(s * A_SUP, (s + 1) * A_SUP, fix, 0)

    def drain_scatter(s, rbuf):
        for j in range(A_SUP):
            pltpu.make_async_copy(
                rbuf.at[j], table.at[cidx.at[s * A_SUP + j]], ssem).wait()

    def do_super(s, rbuf, rbuf_prev):
        gds = [
            pltpu.async_copy(
                xcat_hbm.at[ridx.at[s * A_SUP + j]], rbuf.at[j], gsem)
            for j in range(A_SUP)
        ]

        @pl.when(s < A_NSUP - 1)
        def _():
            fix_sup(s + 1)

        @pl.when(s > 0)
        def _():
            drain_scatter(s - 1, rbuf_prev)

        for d in gds:
            d.wait()
        for j in range(A_SUP):
            pltpu.async_copy(
                rbuf.at[j], table.at[cidx.at[s * A_SUP + j]], ssem, add=True)

    for ph in range(A_PH):
        rbase = sid * A_ROWS + ph * A_PROWS
        pltpu.sync_copy(ei_hbm.at[0, pl.ds(rbase, A_PROWS)], ridx)
        pltpu.sync_copy(ei_hbm.at[1, pl.ds(rbase, A_PROWS)], cidx)
        fix_sup(0)

        def pair(h, _):
            do_super(2 * h, rbufA, rbufB)
            do_super(2 * h + 1, rbufB, rbufA)
            return 0
        lax.fori_loop(0, (A_NSUP - 1) // 2, pair, 0)
        last = A_NSUP - 1
        gds = [
            pltpu.async_copy(
                xcat_hbm.at[ridx.at[last * A_SUP + j]], rbufA.at[j], gsem)
            for j in range(A_SUP)
        ]
        drain_scatter(last - 1, rbufB)
        for d in gds:
            d.wait()
        for j in range(A_SUP):
            pltpu.async_copy(
                rbufA.at[j], table.at[cidx.at[last * A_SUP + j]], ssem,
                add=True)
        drain_scatter(last, rbufA)

    plsc.subcore_barrier()

    def piece(p, _):
        start = sid * STRIPE + p * CHUNK

        @pl.when(start < N)
        def _():
            pltpu.sync_copy(table.at[pl.ds(start, CHUNK)], ebuf)
            pltpu.sync_copy(y2_hbm.at[cid, pl.ds(start, CHUNK)], xbuf)
            pltpu.sync_copy(dis_hbm.at[pl.ds(start, CHUNK)], dbuf)
            for g in range(CHUNK // LANES):
                dv = dbuf[pl.ds(g * LANES, LANES)]
                for j in range(LANES):
                    r = g * LANES + j
                    d = jnp.broadcast_to(dv[j], (LANES,))
                    for q in range(DH // LANES):
                        sl = pl.ds(q * LANES, LANES)
                        ebuf[r, sl] = d * ebuf[r, sl] + xbuf[r, sl]
            pltpu.sync_copy(
                ebuf, out_hbm.at[pl.ds(start, CHUNK), pl.ds(cid * DH, DH)])
        return 0
    lax.fori_loop(0, STRIPE // CHUNK, piece, 0)


_agg_call = pl.kernel(
    _agg_body,
    out_type=jax.ShapeDtypeStruct((N, D), _f32),
    mesh=_mesh,
    scratch_types=[
        pltpu.VMEM_SHARED((TAB_ROWS, DH), _f32),
        pltpu.VMEM((A_PROWS, CHUNK), jnp.int32),
        pltpu.VMEM((A_PROWS, CHUNK), jnp.int32),
        pltpu.VMEM((A_SUP, CHUNK, DH), _f32),
        pltpu.VMEM((A_SUP, CHUNK, DH), _f32),
        pltpu.VMEM((CHUNK, DH), _f32),
        pltpu.VMEM((CHUNK, DH), _f32),
        pltpu.VMEM((CHUNK,), _f32),
        pltpu.SemaphoreType.DMA,
        pltpu.SemaphoreType.DMA,
    ],
    compiler_params=_sc_params,
)


_BLK = 1000


def _mm_body(x_ref, w_ref, h_ref, b_ref, xws_ref, y2_ref, dis_ref):
    xw = jnp.dot(x_ref[...], w_ref[...], preferred_element_type=_f32)
    deg = 1.0 + jnp.sum(h_ref[0], axis=-1) + jnp.sum(h_ref[1], axis=-1)
    dis = lax.rsqrt(deg)
    xws = xw * dis[:, None]
    y = xw * (1.0 + 1.0 / deg)[:, None] + b_ref[0][None, :]
    xws_ref[0] = xws[:, :DH]
    xws_ref[1] = xws[:, DH:]
    y2_ref[0] = y[:, :DH]
    y2_ref[1] = y[:, DH:]
    dis_ref[...] = dis[None, None, :]


def _mm_call(x, weight, hist, bias2d):
    return pl.pallas_call(
        _mm_body,
        grid=(N // _BLK,),
        in_specs=[
            pl.BlockSpec((_BLK, D), lambda i: (i, 0)),
            pl.BlockSpec((D, D), lambda i: (0, 0)),
            pl.BlockSpec((NC, _BLK, LANES), lambda i: (0, i, 0)),
            pl.BlockSpec((1, D), lambda i: (0, 0)),
        ],
        out_specs=[
            pl.BlockSpec((2, _BLK, DH), lambda i: (0, i, 0)),
            pl.BlockSpec((2, _BLK, DH), lambda i: (0, i, 0)),
            pl.BlockSpec((1, 1, _BLK), lambda i: (i, 0, 0)),
        ],
        out_shape=[
            jax.ShapeDtypeStruct((2, TAB_ROWS, DH), _f32),
            jax.ShapeDtypeStruct((2, TAB_ROWS, DH), _f32),
            jax.ShapeDtypeStruct((N // _BLK, 1, _BLK), _f32),
        ],
    )(x, weight, hist, bias2d)


def kernel(x, edge_index, weight, bias):
    ei3 = edge_index.reshape(2, EROWS, CHUNK)
    hist = _deg_call(ei3)
    xws2, y2, dis_p = _mm_call(x, weight, hist, bias[None, :])
    xcat = xws2.reshape(2 * TAB_ROWS, DH)
    return _agg_call(ei3, xcat, y2, dis_p.reshape(N))


# speedup vs baseline: 40.7192x; 1.1144x over previous
import jax
import jax.numpy as jnp
from jax import lax
from jax.experimental import pallas as pl
from jax.experimental.pallas import tpu as pltpu
from jax.experimental.pallas import tpu_sc as plsc

N = 10000
E = 320000
D = 128
DH = D // 2

NC = 2
NS = 16
LANES = 16

CHUNK = 80
EROWS = E // CHUNK

TAB_ROWS = 10240
STRIPE = TAB_ROWS // NS
TRASH = 10100

_mesh = plsc.VectorSubcoreMesh(
    core_axis_name="c", subcore_axis_name="s", num_cores=NC, num_subcores=NS)

_sc_params = pltpu.CompilerParams(use_tc_tiling_on_sc=False)

_f32 = jnp.float32


def _zero16():
    return jnp.broadcast_to(jnp.float32(0.0), (LANES,))


H_ROWS = EROWS // (NC * NS)
H_SUP = 25
H_NSUP = H_ROWS // H_SUP


def _deg_body(ei_hbm, out_hbm, table, ridx, cidx, ones_v, zbuf, ssem):
    cid = lax.axis_index("c")
    sid = lax.axis_index("s")

    lane = lax.iota(jnp.int32, LANES)
    one_hot = jnp.where(lane == 0, jnp.float32(1.0), jnp.float32(0.0))

    def zfill(i, _):
        zbuf[i, :] = _zero16()
        return 0
    lax.fori_loop(0, STRIPE, zfill, 0)

    def ofill(i, _):
        ones_v[i, :] = one_hot
        return 0
    lax.fori_loop(0, CHUNK, ofill, 0)

    pltpu.sync_copy(zbuf, table.at[pl.ds(sid * STRIPE, STRIPE)])
    plsc.subcore_barrier()

    rbase = (cid * NS + sid) * H_ROWS
    pltpu.sync_copy(ei_hbm.at[0, pl.ds(rbase, H_ROWS)], ridx)
    pltpu.sync_copy(ei_hbm.at[1, pl.ds(rbase, H_ROWS)], cidx)

    def fix_sup(s):
        def fix(row, _):
            for i in range(CHUNK // LANES):
                sl = pl.ds(i * LANES, LANES)
                r = ridx[row, sl]
                c = cidx[row, sl]
                ridx[row, sl] = jnp.where(r == c, jnp.int32(TRASH), r)
            return 0
        lax.fori_loop(s * H_SUP, (s + 1) * H_SUP, fix, 0)

    def drain(s):
        for j in range(H_SUP):
            pltpu.make_async_copy(
                ones_v, table.at[ridx.at[s * H_SUP + j]], ssem).wait()

    fix_sup(0)

    def loop(s, _):
        for j in range(H_SUP):
            pltpu.async_copy(
                ones_v, table.at[ridx.at[s * H_SUP + j]], ssem, add=True)

        @pl.when(s < H_NSUP - 1)
        def _():
            fix_sup(s + 1)

        @pl.when(s > 0)
        def _():
            drain(s - 1)
        return 0
    lax.fori_loop(0, H_NSUP, loop, 0)
    drain(H_NSUP - 1)

    plsc.subcore_barrier()
    pltpu.sync_copy(table.at[pl.ds(sid * STRIPE, STRIPE)],
                    out_hbm.at[cid, pl.ds(sid * STRIPE, STRIPE)])


_deg_call = pl.kernel(
    _deg_body,
    out_type=jax.ShapeDtypeStruct((NC, TAB_ROWS, LANES), _f32),
    mesh=_mesh,
    scratch_types=[
        pltpu.VMEM_SHARED((TAB_ROWS, LANES), _f32),
        pltpu.VMEM((H_ROWS, CHUNK), jnp.int32),
        pltpu.VMEM((H_ROWS, CHUNK), jnp.int32),
        pltpu.VMEM((CHUNK, LANES), _f32),
        pltpu.VMEM((STRIPE, LANES), _f32),
        pltpu.SemaphoreType.DMA,
    ],
    compiler_params=_sc_params,
)


A_ROWS = EROWS // NS
A_PH = 2
A_PROWS = A_ROWS // A_PH
A_SUP = 5
A_NSUP = A_PROWS // A_SUP


def _agg_body(ei_hbm, xcat_hbm, y2_hbm, dis_hbm, out_hbm,
              table, ridx, cidx, rbufA, rbufB,
              ebuf, xbuf, dbuf, gsem, ssem):
    cid = lax.axis_index("c")
    sid = lax.axis_index("s")

    def zfill(i, _):
        for j in range(DH // LANES):
            rbufA[0, i, pl.ds(j * LANES, LANES)] = _zero16()
        return 0
    lax.fori_loop(0, CHUNK, zfill, 0)

    for k in range(STRIPE // CHUNK):
        pltpu.sync_copy(rbufA.at[0],
                        table.at[pl.ds(sid * STRIPE + k * CHUNK, CHUNK)])
    plsc.subcore_barrier()

    def fix_sup(s):
        def fix(row, _):
            for i in range(CHUNK // LANES):
                sl = pl.ds(i * LANES, LANES)
                r = ridx[row, sl]
                c = cidx[row, sl]
                cidx[row, sl] = jnp.where(r == c, jnp.int32(TRASH), c)
                ridx[row, sl] = r + r + cid
            return 0
        lax.fori_loop(s * A_SUP, (s + 1) * A_SUP, fix, 0)

    def drain_scatter(s, rbuf):
        for j in range(A_SUP):
            pltpu.make_async_copy(
                rbuf.at[j], table.at[cidx.at[s * A_SUP + j]], ssem).wait()

    def do_super(s, rbuf, rbuf_prev):
        gds = [
            pltpu.async_copy(
                xcat_hbm.at[ridx.at[s * A_SUP + j]], rbuf.at[j], gsem)
            for j in range(A_SUP)
        ]

        @pl.when(s < A_NSUP - 1)
        def _():
            fix_sup(s + 1)

        @pl.when(s > 0)
        def _():
            drain_scatter(s - 1, rbuf_prev)

        for d in gds:
            d.wait()
        for j in range(A_SUP):
            pltpu.async_copy(
                rbuf.at[j], table.at[cidx.at[s * A_SUP + j]], ssem, add=True)

    for ph in range(A_PH):
        rbase = sid * A_ROWS + ph * A_PROWS
        pltpu.sync_copy(ei_hbm.at[0, pl.ds(rbase, A_PROWS)], ridx)
        pltpu.sync_copy(ei_hbm.at[1, pl.ds(rbase, A_PROWS)], cidx)
        fix_sup(0)

        def pair(h, _):
            do_super(2 * h, rbufA, rbufB)
            do_super(2 * h + 1, rbufB, rbufA)
            return 0
        lax.fori_loop(0, (A_NSUP - 1) // 2, pair, 0)
        last = A_NSUP - 1
        gds = [
            pltpu.async_copy(
                xcat_hbm.at[ridx.at[last * A_SUP + j]], rbufA.at[j], gsem)
            for j in range(A_SUP)
        ]
        drain_scatter(last - 1, rbufB)
        for d in gds:
            d.wait()
        for j in range(A_SUP):
            pltpu.async_copy(
                rbufA.at[j], table.at[cidx.at[last * A_SUP + j]], ssem,
                add=True)
        drain_scatter(last, rbufA)

    plsc.subcore_barrier()

    def piece(p, _):
        start = sid * STRIPE + p * CHUNK

        @pl.when(start < N)
        def _():
            pltpu.sync_copy(table.at[pl.ds(start, CHUNK)], ebuf)
            pltpu.sync_copy(
                y2_hbm.at[pl.ds(start, CHUNK), pl.ds(cid * DH, DH)], xbuf)
            pltpu.sync_copy(dis_hbm.at[pl.ds(start, CHUNK)], dbuf)
            for g in range(CHUNK // LANES):
                dv = dbuf[pl.ds(g * LANES, LANES)]
                for j in range(LANES):
                    r = g * LANES + j
                    d = jnp.broadcast_to(dv[j], (LANES,))
                    for q in range(DH // LANES):
                        sl = pl.ds(q * LANES, LANES)
                        ebuf[r, sl] = d * ebuf[r, sl] + xbuf[r, sl]
            pltpu.sync_copy(
                ebuf, out_hbm.at[pl.ds(start, CHUNK), pl.ds(cid * DH, DH)])
        return 0
    lax.fori_loop(0, STRIPE // CHUNK, piece, 0)


_agg_call = pl.kernel(
    _agg_body,
    out_type=jax.ShapeDtypeStruct((N, D), _f32),
    mesh=_mesh,
    scratch_types=[
        pltpu.VMEM_SHARED((TAB_ROWS, DH), _f32),
        pltpu.VMEM((A_PROWS, CHUNK), jnp.int32),
        pltpu.VMEM((A_PROWS, CHUNK), jnp.int32),
        pltpu.VMEM((A_SUP, CHUNK, DH), _f32),
        pltpu.VMEM((A_SUP, CHUNK, DH), _f32),
        pltpu.VMEM((CHUNK, DH), _f32),
        pltpu.VMEM((CHUNK, DH), _f32),
        pltpu.VMEM((CHUNK,), _f32),
        pltpu.SemaphoreType.DMA,
        pltpu.SemaphoreType.DMA,
    ],
    compiler_params=_sc_params,
)


_BLK = 1024


def _mm_body(x_ref, w_ref, h_ref, b_ref, xws_ref, y_ref, dis_ref):
    xw = jnp.dot(x_ref[...], w_ref[...], preferred_element_type=_f32)
    deg = 1.0 + jnp.sum(h_ref[0], axis=-1) + jnp.sum(h_ref[1], axis=-1)
    dis = lax.rsqrt(deg)
    xws_ref[...] = xw * dis[:, None]
    y_ref[...] = xw * (1.0 + 1.0 / deg)[:, None] + b_ref[0][None, :]
    dis_ref[...] = dis[None, None, :]


def _mm_call(x, weight, hist, bias2d):
    return pl.pallas_call(
        _mm_body,
        grid=(TAB_ROWS // _BLK,),
        in_specs=[
            pl.BlockSpec((_BLK, D), lambda i: (i, 0)),
            pl.BlockSpec((D, D), lambda i: (0, 0)),
            pl.BlockSpec((NC, _BLK, LANES), lambda i: (0, i, 0)),
            pl.BlockSpec((1, D), lambda i: (0, 0)),
        ],
        out_specs=[
            pl.BlockSpec((_BLK, D), lambda i: (i, 0)),
            pl.BlockSpec((_BLK, D), lambda i: (i, 0)),
            pl.BlockSpec((1, 1, _BLK), lambda i: (i, 0, 0)),
        ],
        out_shape=[
            jax.ShapeDtypeStruct((TAB_ROWS, D), _f32),
            jax.ShapeDtypeStruct((TAB_ROWS, D), _f32),
            jax.ShapeDtypeStruct((TAB_ROWS // _BLK, 1, _BLK), _f32),
        ],
    )(x, weight, hist, bias2d)


def kernel(x, edge_index, weight, bias):
    ei3 = edge_index.reshape(2, EROWS, CHUNK)
    hist = _deg_call(ei3)
    xws, y, dis_p = _mm_call(x, weight, hist, bias[None, :])
    xcat = xws.reshape(2 * TAB_ROWS, DH)
    return _agg_call(ei3, xcat, y, dis_p.reshape(TAB_ROWS))


# speedup vs baseline: 42.8505x; 1.0523x over previous
import jax
import jax.numpy as jnp
from jax import lax
from jax.experimental import pallas as pl
from jax.experimental.pallas import tpu as pltpu
from jax.experimental.pallas import tpu_sc as plsc

N = 10000
E = 320000
D = 128
DH = D // 2

NC = 2
NS = 16
LANES = 16

CHUNK = 80
EROWS = E // CHUNK

TAB_ROWS = 10240
STRIPE = TAB_ROWS // NS
TRASH = 10100

_mesh = plsc.VectorSubcoreMesh(
    core_axis_name="c", subcore_axis_name="s", num_cores=NC, num_subcores=NS)

_sc_params = pltpu.CompilerParams(use_tc_tiling_on_sc=False)

_f32 = jnp.float32


def _zero16():
    return jnp.broadcast_to(jnp.float32(0.0), (LANES,))


H_ROWS = EROWS // (NC * NS)
H_SUP = 25
H_NSUP = H_ROWS // H_SUP


def _deg_body(ei_hbm, out_hbm, table, ridx, cidx, ones_v, zbuf, hbuf, ssem):
    cid = lax.axis_index("c")
    sid = lax.axis_index("s")

    lane = lax.iota(jnp.int32, LANES)
    one_hot = jnp.where(lane == 0, jnp.float32(1.0), jnp.float32(0.0))

    def zfill(i, _):
        zbuf[i, :] = _zero16()
        return 0
    lax.fori_loop(0, STRIPE, zfill, 0)

    def ofill(i, _):
        ones_v[i, :] = one_hot
        return 0
    lax.fori_loop(0, CHUNK, ofill, 0)

    pltpu.sync_copy(zbuf, table.at[pl.ds(sid * STRIPE, STRIPE)])
    plsc.subcore_barrier()

    rbase = (cid * NS + sid) * H_ROWS
    pltpu.sync_copy(ei_hbm.at[0, pl.ds(rbase, H_ROWS)], ridx)
    pltpu.sync_copy(ei_hbm.at[1, pl.ds(rbase, H_ROWS)], cidx)

    def fix_sup(s):
        def fix(row, _):
            for i in range(CHUNK // LANES):
                sl = pl.ds(i * LANES, LANES)
                r = ridx[row, sl]
                c = cidx[row, sl]
                ridx[row, sl] = jnp.where(r == c, jnp.int32(TRASH), r)
            return 0
        lax.fori_loop(s * H_SUP, (s + 1) * H_SUP, fix, 0)

    def drain(s):
        for j in range(H_SUP):
            pltpu.make_async_copy(
                ones_v, table.at[ridx.at[s * H_SUP + j]], ssem).wait()

    fix_sup(0)

    def loop(s, _):
        for j in range(H_SUP):
            pltpu.async_copy(
                ones_v, table.at[ridx.at[s * H_SUP + j]], ssem, add=True)

        @pl.when(s < H_NSUP - 1)
        def _():
            fix_sup(s + 1)

        @pl.when(s > 0)
        def _():
            drain(s - 1)
        return 0
    lax.fori_loop(0, H_NSUP, loop, 0)
    drain(H_NSUP - 1)

    plsc.subcore_barrier()
    pltpu.sync_copy(table.at[pl.ds(sid * STRIPE, STRIPE)], zbuf)

    def compact_loop(k, _):
        out16 = _zero16()
        for j in range(LANES):
            row = zbuf[k * LANES + j, :]
            out16 = jnp.where(lane == j, row[0], out16)
        hbuf[pl.ds(k * LANES, LANES)] = out16
        return 0
    lax.fori_loop(0, STRIPE // LANES, compact_loop, 0)
    pltpu.sync_copy(hbuf, out_hbm.at[cid, pl.ds(sid * STRIPE, STRIPE)])


_deg_call = pl.kernel(
    _deg_body,
    out_type=jax.ShapeDtypeStruct((NC, TAB_ROWS), _f32),
    mesh=_mesh,
    scratch_types=[
        pltpu.VMEM_SHARED((TAB_ROWS, LANES), _f32),
        pltpu.VMEM((H_ROWS, CHUNK), jnp.int32),
        pltpu.VMEM((H_ROWS, CHUNK), jnp.int32),
        pltpu.VMEM((CHUNK, LANES), _f32),
        pltpu.VMEM((STRIPE, LANES), _f32),
        pltpu.VMEM((STRIPE,), _f32),
        pltpu.SemaphoreType.DMA,
    ],
    compiler_params=_sc_params,
)


A_ROWS = EROWS // NS
A_PH = 2
A_PROWS = A_ROWS // A_PH
A_SUP = 5
A_NSUP = A_PROWS // A_SUP


def _agg_body(ei_hbm, xcat_hbm, y2_hbm, dis_hbm, out_hbm,
              table, ridx, cidx, rbufA, rbufB,
              ebuf, xbuf, dbuf, gsem, ssem):
    cid = lax.axis_index("c")
    sid = lax.axis_index("s")

    def zfill(i, _):
        for j in range(DH // LANES):
            rbufA[0, i, pl.ds(j * LANES, LANES)] = _zero16()
        return 0
    lax.fori_loop(0, CHUNK, zfill, 0)

    for k in range(STRIPE // CHUNK):
        pltpu.sync_copy(rbufA.at[0],
                        table.at[pl.ds(sid * STRIPE + k * CHUNK, CHUNK)])
    plsc.subcore_barrier()

    def fix_sup(s):
        def fix(row, _):
            for i in range(CHUNK // LANES):
                sl = pl.ds(i * LANES, LANES)
                r = ridx[row, sl]
                c = cidx[row, sl]
                cidx[row, sl] = jnp.where(r == c, jnp.int32(TRASH), c)
                ridx[row, sl] = r + r + cid
            return 0
        lax.fori_loop(s * A_SUP, (s + 1) * A_SUP, fix, 0)

    def drain_scatter(s, rbuf):
        for j in range(A_SUP):
            pltpu.make_async_copy(
                rbuf.at[j], table.at[cidx.at[s * A_SUP + j]], ssem).wait()

    def do_super(s, rbuf, rbuf_prev):
        gds = [
            pltpu.async_copy(
                xcat_hbm.at[ridx.at[s * A_SUP + j]], rbuf.at[j], gsem)
            for j in range(A_SUP)
        ]

        @pl.when(s < A_NSUP - 1)
        def _():
            fix_sup(s + 1)

        @pl.when(s > 0)
        def _():
            drain_scatter(s - 1, rbuf_prev)

        for d in gds:
            d.wait()
        for j in range(A_SUP):
            pltpu.async_copy(
                rbuf.at[j], table.at[cidx.at[s * A_SUP + j]], ssem, add=True)

    for ph in range(A_PH):
        rbase = sid * A_ROWS + ph * A_PROWS
        pltpu.sync_copy(ei_hbm.at[0, pl.ds(rbase, A_PROWS)], ridx)
        pltpu.sync_copy(ei_hbm.at[1, pl.ds(rbase, A_PROWS)], cidx)
        fix_sup(0)

        def pair(h, _):
            do_super(2 * h, rbufA, rbufB)
            do_super(2 * h + 1, rbufB, rbufA)
            return 0
        lax.fori_loop(0, (A_NSUP - 1) // 2, pair, 0)
        last = A_NSUP - 1
        gds = [
            pltpu.async_copy(
                xcat_hbm.at[ridx.at[last * A_SUP + j]], rbufA.at[j], gsem)
            for j in range(A_SUP)
        ]
        drain_scatter(last - 1, rbufB)
        for d in gds:
            d.wait()
        for j in range(A_SUP):
            pltpu.async_copy(
                rbufA.at[j], table.at[cidx.at[last * A_SUP + j]], ssem,
                add=True)
        drain_scatter(last, rbufA)

    plsc.subcore_barrier()

    def piece(p, _):
        start = sid * STRIPE + p * CHUNK

        @pl.when(start < N)
        def _():
            pltpu.sync_copy(table.at[pl.ds(start, CHUNK)], ebuf)
            pltpu.sync_copy(
                y2_hbm.at[pl.ds(start, CHUNK), pl.ds(cid * DH, DH)], xbuf)
            pltpu.sync_copy(dis_hbm.at[pl.ds(start, CHUNK)], dbuf)
            for g in range(CHUNK // LANES):
                dv = dbuf[pl.ds(g * LANES, LANES)]
                for j in range(LANES):
                    r = g * LANES + j
                    d = jnp.broadcast_to(dv[j], (LANES,))
                    for q in range(DH // LANES):
                        sl = pl.ds(q * LANES, LANES)
                        ebuf[r, sl] = d * ebuf[r, sl] + xbuf[r, sl]
            pltpu.sync_copy(
                ebuf, out_hbm.at[pl.ds(start, CHUNK), pl.ds(cid * DH, DH)])
        return 0
    lax.fori_loop(0, STRIPE // CHUNK, piece, 0)


_agg_call = pl.kernel(
    _agg_body,
    out_type=jax.ShapeDtypeStruct((N, D), _f32),
    mesh=_mesh,
    scratch_types=[
        pltpu.VMEM_SHARED((TAB_ROWS, DH), _f32),
        pltpu.VMEM((A_PROWS, CHUNK), jnp.int32),
        pltpu.VMEM((A_PROWS, CHUNK), jnp.int32),
        pltpu.VMEM((A_SUP, CHUNK, DH), _f32),
        pltpu.VMEM((A_SUP, CHUNK, DH), _f32),
        pltpu.VMEM((CHUNK, DH), _f32),
        pltpu.VMEM((CHUNK, DH), _f32),
        pltpu.VMEM((CHUNK,), _f32),
        pltpu.SemaphoreType.DMA,
        pltpu.SemaphoreType.DMA,
    ],
    compiler_params=_sc_params,
)


_BLK = 1024


def _mm_body(x_ref, w_ref, h_ref, b_ref, xws_ref, y_ref, dis_ref):
    xw = jnp.dot(x_ref[...], w_ref[...], preferred_element_type=_f32)
    deg = 1.0 + h_ref[0] + h_ref[1]
    dis = lax.rsqrt(deg)
    xws_ref[...] = xw * dis[:, None]
    y_ref[...] = xw * (1.0 + 1.0 / deg)[:, None] + b_ref[0][None, :]
    dis_ref[...] = dis[None, None, :]


def _mm_call(x, weight, hist, bias2d):
    return pl.pallas_call(
        _mm_body,
        grid=(TAB_ROWS // _BLK,),
        in_specs=[
            pl.BlockSpec((_BLK, D), lambda i: (i, 0)),
            pl.BlockSpec((D, D), lambda i: (0, 0)),
            pl.BlockSpec((NC, _BLK), lambda i: (0, i)),
            pl.BlockSpec((1, D), lambda i: (0, 0)),
        ],
        out_specs=[
            pl.BlockSpec((_BLK, D), lambda i: (i, 0)),
            pl.BlockSpec((_BLK, D), lambda i: (i, 0)),
            pl.BlockSpec((1, 1, _BLK), lambda i: (i, 0, 0)),
        ],
        out_shape=[
            jax.ShapeDtypeStruct((TAB_ROWS, D), _f32),
            jax.ShapeDtypeStruct((TAB_ROWS, D), _f32),
            jax.ShapeDtypeStruct((TAB_ROWS // _BLK, 1, _BLK), _f32),
        ],
    )(x, weight, hist, bias2d)


def kernel(x, edge_index, weight, bias):
    ei3 = edge_index.reshape(2, EROWS, CHUNK)
    hist = _deg_call(ei3)
    xws, y, dis_p = _mm_call(x, weight, hist, bias[None, :])
    xcat = xws.reshape(2 * TAB_ROWS, DH)
    return _agg_call(ei3, xcat, y, dis_p.reshape(TAB_ROWS))


# speedup vs baseline: 43.5142x; 1.0155x over previous
import jax
import jax.numpy as jnp
from jax import lax
from jax.experimental import pallas as pl
from jax.experimental.pallas import tpu as pltpu
from jax.experimental.pallas import tpu_sc as plsc

N = 10000
E = 320000
D = 128
DH = D // 2

NC = 2
NS = 16
LANES = 16

CHUNK = 80
EROWS = E // CHUNK

TAB_ROWS = 10240
STRIPE = TAB_ROWS // NS
TRASH = 10100

_mesh = plsc.VectorSubcoreMesh(
    core_axis_name="c", subcore_axis_name="s", num_cores=NC, num_subcores=NS)

_sc_params = pltpu.CompilerParams(use_tc_tiling_on_sc=False)

_f32 = jnp.float32


def _zero16():
    return jnp.broadcast_to(jnp.float32(0.0), (LANES,))


H_ROWS = EROWS // (NC * NS)
H_SUP = 25
H_NSUP = H_ROWS // H_SUP


def _deg_body(ei_hbm, out_hbm, table, ridx, cidx, ones_v, zbuf, hbuf, ssem):
    cid = lax.axis_index("c")
    sid = lax.axis_index("s")

    lane = lax.iota(jnp.int32, LANES)
    one_hot = jnp.where(lane == 0, jnp.float32(1.0), jnp.float32(0.0))

    def zfill(i, _):
        zbuf[i, :] = _zero16()
        return 0
    lax.fori_loop(0, STRIPE, zfill, 0)

    def ofill(i, _):
        ones_v[i, :] = one_hot
        return 0
    lax.fori_loop(0, CHUNK, ofill, 0)

    pltpu.sync_copy(zbuf, table.at[pl.ds(sid * STRIPE, STRIPE)])
    plsc.subcore_barrier()

    rbase = (cid * NS + sid) * H_ROWS
    pltpu.sync_copy(ei_hbm.at[0, pl.ds(rbase, H_ROWS)], ridx)
    pltpu.sync_copy(ei_hbm.at[1, pl.ds(rbase, H_ROWS)], cidx)

    def fix_sup(s):
        def fix(row, _):
            for i in range(CHUNK // LANES):
                sl = pl.ds(i * LANES, LANES)
                r = ridx[row, sl]
                c = cidx[row, sl]
                ridx[row, sl] = jnp.where(r == c, jnp.int32(TRASH), r)
            return 0
        lax.fori_loop(s * H_SUP, (s + 1) * H_SUP, fix, 0)

    def drain(s):
        for j in range(H_SUP):
            pltpu.make_async_copy(
                ones_v, table.at[ridx.at[s * H_SUP + j]], ssem).wait()

    fix_sup(0)

    def loop(s, _):
        for j in range(H_SUP):
            pltpu.async_copy(
                ones_v, table.at[ridx.at[s * H_SUP + j]], ssem, add=True)

        @pl.when(s < H_NSUP - 1)
        def _():
            fix_sup(s + 1)

        @pl.when(s > 0)
        def _():
            drain(s - 1)
        return 0
    lax.fori_loop(0, H_NSUP, loop, 0)
    drain(H_NSUP - 1)

    plsc.subcore_barrier()
    pltpu.sync_copy(table.at[pl.ds(sid * STRIPE, STRIPE)], zbuf)

    def compact_loop(k, _):
        out16 = _zero16()
        for j in range(LANES):
            row = zbuf[k * LANES + j, :]
            out16 = jnp.where(lane == j, row[0], out16)
        hbuf[pl.ds(k * LANES, LANES)] = out16
        return 0
    lax.fori_loop(0, STRIPE // LANES, compact_loop, 0)
    pltpu.sync_copy(hbuf, out_hbm.at[cid, pl.ds(sid * STRIPE, STRIPE)])


_deg_call = pl.kernel(
    _deg_body,
    out_type=jax.ShapeDtypeStruct((NC, TAB_ROWS), _f32),
    mesh=_mesh,
    scratch_types=[
        pltpu.VMEM_SHARED((TAB_ROWS, LANES), _f32),
        pltpu.VMEM((H_ROWS, CHUNK), jnp.int32),
        pltpu.VMEM((H_ROWS, CHUNK), jnp.int32),
        pltpu.VMEM((CHUNK, LANES), _f32),
        pltpu.VMEM((STRIPE, LANES), _f32),
        pltpu.VMEM((STRIPE,), _f32),
        pltpu.SemaphoreType.DMA,
    ],
    compiler_params=_sc_params,
)


A_ROWS = EROWS // NS
A_PH = 2
A_PROWS = A_ROWS // A_PH
A_SUP = 5
A_NSUP = A_PROWS // A_SUP


def _agg_body(ei_hbm, xcat_hbm, y2_hbm, dis_hbm, out_hbm,
              table, ridx, cidx, rbufA, rbufB,
              ebuf, xbuf, dbuf, gsem, ssem):
    cid = lax.axis_index("c")
    sid = lax.axis_index("s")

    def zfill(i, _):
        for j in range(DH // LANES):
            rbufA[0, i, pl.ds(j * LANES, LANES)] = _zero16()
        return 0
    lax.fori_loop(0, CHUNK, zfill, 0)

    for k in range(STRIPE // CHUNK):
        pltpu.sync_copy(rbufA.at[0],
                        table.at[pl.ds(sid * STRIPE + k * CHUNK, CHUNK)])
    plsc.subcore_barrier()

    def fix_sup(s):
        def fix(row, _):
            for i in range(CHUNK // LANES):
                sl = pl.ds(i * LANES, LANES)
                r = ridx[row, sl]
                c = cidx[row, sl]
                cidx[row, sl] = jnp.where(r == c, jnp.int32(TRASH), c)
                ridx[row, sl] = r + r + cid
            return 0
        lax.fori_loop(s * A_SUP, (s + 1) * A_SUP, fix, 0)

    def drain_scatter(s, rbuf):
        for j in range(A_SUP):
            pltpu.make_async_copy(
                rbuf.at[j], table.at[cidx.at[s * A_SUP + j]], ssem).wait()

    def do_super(s, rbuf, rbuf_prev):
        gds = [
            pltpu.async_copy(
                xcat_hbm.at[ridx.at[s * A_SUP + j]], rbuf.at[j], gsem)
            for j in range(A_SUP)
        ]

        @pl.when(s < A_NSUP - 1)
        def _():
            fix_sup(s + 1)

        @pl.when(s > 0)
        def _():
            drain_scatter(s - 1, rbuf_prev)

        for d in gds:
            d.wait()
        for j in range(A_SUP):
            pltpu.async_copy(
                rbuf.at[j], table.at[cidx.at[s * A_SUP + j]], ssem, add=True)

    for ph in range(A_PH):
        rbase = sid * A_ROWS + ph * A_PROWS
        pltpu.sync_copy(ei_hbm.at[0, pl.ds(rbase, A_PROWS)], ridx)
        pltpu.sync_copy(ei_hbm.at[1, pl.ds(rbase, A_PROWS)], cidx)
        fix_sup(0)

        def pair(h, _):
            do_super(2 * h, rbufA, rbufB)
            do_super(2 * h + 1, rbufB, rbufA)
            return 0
        lax.fori_loop(0, (A_NSUP - 1) // 2, pair, 0)
        last = A_NSUP - 1
        gds = [
            pltpu.async_copy(
                xcat_hbm.at[ridx.at[last * A_SUP + j]], rbufA.at[j], gsem)
            for j in range(A_SUP)
        ]
        drain_scatter(last - 1, rbufB)
        for d in gds:
            d.wait()
        for j in range(A_SUP):
            pltpu.async_copy(
                rbufA.at[j], table.at[cidx.at[last * A_SUP + j]], ssem,
                add=True)
        drain_scatter(last, rbufA)

    plsc.subcore_barrier()

    def piece(p, _):
        start = sid * STRIPE + p * CHUNK

        @pl.when(start < N)
        def _():
            pltpu.sync_copy(table.at[pl.ds(start, CHUNK)], ebuf)
            pltpu.sync_copy(
                y2_hbm.at[pl.ds(start, CHUNK), pl.ds(cid * DH, DH)], xbuf)
            pltpu.sync_copy(dis_hbm.at[pl.ds(start, CHUNK)], dbuf)
            for g in range(CHUNK // LANES):
                dv = dbuf[pl.ds(g * LANES, LANES)]
                for j in range(LANES):
                    r = g * LANES + j
                    d = jnp.broadcast_to(dv[j], (LANES,))
                    for q in range(DH // LANES):
                        sl = pl.ds(q * LANES, LANES)
                        ebuf[r, sl] = d * ebuf[r, sl] + xbuf[r, sl]
            pltpu.sync_copy(
                ebuf, out_hbm.at[pl.ds(start, CHUNK), pl.ds(cid * DH, DH)])
        return 0
    lax.fori_loop(0, STRIPE // CHUNK, piece, 0)


_agg_call = pl.kernel(
    _agg_body,
    out_type=jax.ShapeDtypeStruct((N, D), _f32),
    mesh=_mesh,
    scratch_types=[
        pltpu.VMEM_SHARED((TAB_ROWS, DH), _f32),
        pltpu.VMEM((A_PROWS, CHUNK), jnp.int32),
        pltpu.VMEM((A_PROWS, CHUNK), jnp.int32),
        pltpu.VMEM((A_SUP, CHUNK, DH), _f32),
        pltpu.VMEM((A_SUP, CHUNK, DH), _f32),
        pltpu.VMEM((CHUNK, DH), _f32),
        pltpu.VMEM((CHUNK, DH), _f32),
        pltpu.VMEM((CHUNK,), _f32),
        pltpu.SemaphoreType.DMA,
        pltpu.SemaphoreType.DMA,
    ],
    compiler_params=_sc_params,
)


_BLK = 2048


def _mm_body(x_ref, w_ref, h_ref, b_ref, xws_ref, y_ref, dis_ref):
    xw = jnp.dot(x_ref[...], w_ref[...], preferred_element_type=_f32)
    deg = 1.0 + h_ref[0] + h_ref[1]
    dis = lax.rsqrt(deg)
    xws_ref[...] = xw * dis[:, None]
    y_ref[...] = xw * (1.0 + 1.0 / deg)[:, None] + b_ref[0][None, :]
    dis_ref[...] = dis[None, None, :]


def _mm_call(x, weight, hist, bias2d):
    return pl.pallas_call(
        _mm_body,
        grid=(TAB_ROWS // _BLK,),
        in_specs=[
            pl.BlockSpec((_BLK, D), lambda i: (i, 0)),
            pl.BlockSpec((D, D), lambda i: (0, 0)),
            pl.BlockSpec((NC, _BLK), lambda i: (0, i)),
            pl.BlockSpec((1, D), lambda i: (0, 0)),
        ],
        out_specs=[
            pl.BlockSpec((_BLK, D), lambda i: (i, 0)),
            pl.BlockSpec((_BLK, D), lambda i: (i, 0)),
            pl.BlockSpec((1, 1, _BLK), lambda i: (i, 0, 0)),
        ],
        out_shape=[
            jax.ShapeDtypeStruct((TAB_ROWS, D), _f32),
            jax.ShapeDtypeStruct((TAB_ROWS, D), _f32),
            jax.ShapeDtypeStruct((TAB_ROWS // _BLK, 1, _BLK), _f32),
        ],
    )(x, weight, hist, bias2d)


def kernel(x, edge_index, weight, bias):
    ei3 = edge_index.reshape(2, EROWS, CHUNK)
    hist = _deg_call(ei3)
    xws, y, dis_p = _mm_call(x, weight, hist, bias[None, :])
    xcat = xws.reshape(2 * TAB_ROWS, DH)
    return _agg_call(ei3, xcat, y, dis_p.reshape(TAB_ROWS))
